# baseline (device time: 101125 ns/iter reference)
import jax
import jax.numpy as jnp
from jax import lax
from jax.experimental import pallas as pl
from jax.experimental.pallas import tpu as pltpu

KB = 2048
CHUNK = 512
HALF = CHUNK // 2
P = 8

_S_Z, _S_X1, _S_Y1, _S_X2, _S_Y2 = range(5)


def kernel(dy, W):
    m, k = dy.shape
    d, k2 = W.shape
    assert k == k2
    nk = k // KB
    cp = d // P

    c_out = 2 * lax.axis_index("x") + lax.axis_index("y")

    def body(c_ref, dyc_ref, w_ref, out_ref, mine, sendz, zrecv, ssem, rsem):
        h = pl.program_id(0)
        ki = pl.program_id(1)

        my_x = lax.axis_index("x")
        my_y = lax.axis_index("y")
        my_z = lax.axis_index("z")
        nz = (my_x, my_y, 1 - my_z)
        nx = (1 - my_x, my_y, my_z)
        ny = (my_x, 1 - my_y, my_z)
        c = 2 * my_x + my_y
        cx = 2 * (1 - my_x) + my_y
        cy = 2 * my_x + (1 - my_y)
        cd = 2 * (1 - my_x) + (1 - my_y)

        def rd(src, dst, kind, piece, dev):
            return pltpu.make_async_remote_copy(
                src_ref=src, dst_ref=dst,
                send_sem=ssem.at[kind, piece], recv_sem=rsem.at[kind, piece],
                device_id=dev, device_id_type=pl.DeviceIdType.MESH)

        def own_slice(p):
            return out_ref.at[pl.ds(c * CHUNK, CHUNK), pl.ds(p * cp, cp)]

        def zreduce_and_gather(p):
            rd(sendz.at[p], zrecv.at[p], _S_Z, p, nz).wait()
            mine[p] += zrecv[p].astype(jnp.float32)
            out_ref[pl.ds(c * CHUNK, CHUNK), pl.ds(p * cp, cp)] = (
                mine[p].astype(jnp.bfloat16))
            rd(own_slice(p), own_slice(p), _S_X1, p, nx).start()
            rd(own_slice(p), own_slice(p), _S_Y1, p, ny).start()

        @pl.when(jnp.logical_and(h == 0, ki == 0))
        def _():
            barrier = pltpu.get_barrier_semaphore()
            for nbr in (nz, nx, ny):
                pl.semaphore_signal(barrier, inc=1, device_id=nbr,
                                    device_id_type=pl.DeviceIdType.MESH)
            pl.semaphore_wait(barrier, 3)

        @pl.when(ki == 0)
        def _():
            mine[h] = jnp.zeros_like(mine[h])

        mine[h] += lax.dot_general(
            dyc_ref[...], w_ref[...],
            dimension_numbers=(((1,), (1,)), ((), ())),
            preferred_element_type=jnp.float32,
        )

        @pl.when(ki == nk - 1)
        def _():
            sendz[h] = mine[h].astype(jnp.bfloat16)
            rd(sendz.at[h], zrecv.at[h], _S_Z, h, nz).start()

            @pl.when(h > 0)
            def _():
                zreduce_and_gather(h - 1)

        @pl.when(jnp.logical_and(h == P - 1, ki == nk - 1))
        def _():
            zreduce_and_gather(P - 1)

            for p in range(P):
                rd(own_slice(p), own_slice(p), _S_X1, p, nx).wait()
                rd(own_slice(p), own_slice(p), _S_Y1, p, ny).wait()
                fx = out_ref.at[pl.ds(cy * CHUNK, HALF), pl.ds(p * cp, cp)]
                fy = out_ref.at[pl.ds(cx * CHUNK + HALF, HALF),
                                pl.ds(p * cp, cp)]
                rd(fx, fx, _S_X2, p, nx).start()
                rd(fy, fy, _S_Y2, p, ny).start()

            for p in range(P):
                fx = out_ref.at[pl.ds(cy * CHUNK, HALF), pl.ds(p * cp, cp)]
                fy = out_ref.at[pl.ds(cx * CHUNK + HALF, HALF),
                                pl.ds(p * cp, cp)]
                rd(fx, fx, _S_X2, p, nx).wait()
                rd(fy, fy, _S_Y2, p, ny).wait()

    grid_spec = pltpu.PrefetchScalarGridSpec(
        num_scalar_prefetch=1,
        grid=(P, nk),
        in_specs=[
            pl.BlockSpec((CHUNK, KB), lambda h, ki, c_ref: (c_ref[0], ki)),
            pl.BlockSpec((cp, KB), lambda h, ki, c_ref: (h, ki)),
        ],
        out_specs=pl.BlockSpec((m, d), lambda h, ki, c_ref: (0, 0)),
        scratch_shapes=[
            pltpu.VMEM((P, CHUNK, cp), jnp.float32),
            pltpu.VMEM((P, CHUNK, cp), jnp.bfloat16),
            pltpu.VMEM((P, CHUNK, cp), jnp.bfloat16),
            pltpu.SemaphoreType.DMA((5, P)),
            pltpu.SemaphoreType.DMA((5, P)),
        ],
    )
    return pl.pallas_call(
        body,
        grid_spec=grid_spec,
        out_shape=jax.ShapeDtypeStruct((m, d), jnp.bfloat16),
        compiler_params=pltpu.CompilerParams(
            collective_id=0,
            vmem_limit_bytes=100 * 1024 * 1024,
        ),
    )(jnp.array([c_out], dtype=jnp.int32), dy, W)


# device time: 85421 ns/iter; 1.1838x vs baseline; 1.1838x over previous
import jax
import jax.numpy as jnp
from jax import lax
from jax.experimental import pallas as pl
from jax.experimental.pallas import tpu as pltpu

KB = 4096
CHUNK = 512
HALF = CHUNK // 2
P = 4

_S_Z, _S_X1, _S_Y1, _S_X2, _S_Y2 = range(5)


def kernel(dy, W):
    m, k = dy.shape
    d, k2 = W.shape
    assert k == k2
    nk = k // KB
    cp = d // P

    c_out = 2 * lax.axis_index("x") + lax.axis_index("y")

    def body(c_ref, dyc_ref, w_ref, out_ref, mine, sendz, zrecv, ssem, rsem):
        h = pl.program_id(0)
        ki = pl.program_id(1)

        my_x = lax.axis_index("x")
        my_y = lax.axis_index("y")
        my_z = lax.axis_index("z")
        nz = (my_x, my_y, 1 - my_z)
        nx = (1 - my_x, my_y, my_z)
        ny = (my_x, 1 - my_y, my_z)
        c = 2 * my_x + my_y
        cx = 2 * (1 - my_x) + my_y
        cy = 2 * my_x + (1 - my_y)
        cd = 2 * (1 - my_x) + (1 - my_y)

        def rd(src, dst, kind, piece, dev):
            return pltpu.make_async_remote_copy(
                src_ref=src, dst_ref=dst,
                send_sem=ssem.at[kind, piece], recv_sem=rsem.at[kind, piece],
                device_id=dev, device_id_type=pl.DeviceIdType.MESH)

        def own_slice(p):
            return out_ref.at[pl.ds(c * CHUNK, CHUNK), pl.ds(p * cp, cp)]

        def zreduce_and_gather(p):
            rd(sendz.at[p], zrecv.at[p], _S_Z, p, nz).wait()
            mine[p] += zrecv[p].astype(jnp.float32)
            out_ref[pl.ds(c * CHUNK, CHUNK), pl.ds(p * cp, cp)] = (
                mine[p].astype(jnp.bfloat16))
            rd(own_slice(p), own_slice(p), _S_X1, p, nx).start()
            rd(own_slice(p), own_slice(p), _S_Y1, p, ny).start()

        @pl.when(jnp.logical_and(h == 0, ki == 0))
        def _():
            barrier = pltpu.get_barrier_semaphore()
            for nbr in (nz, nx, ny):
                pl.semaphore_signal(barrier, inc=1, device_id=nbr,
                                    device_id_type=pl.DeviceIdType.MESH)
            pl.semaphore_wait(barrier, 3)

        @pl.when(ki == 0)
        def _():
            mine[h] = jnp.zeros_like(mine[h])

        mine[h] += lax.dot_general(
            dyc_ref[...], w_ref[...],
            dimension_numbers=(((1,), (1,)), ((), ())),
            preferred_element_type=jnp.float32,
        )

        @pl.when(ki == nk - 1)
        def _():
            sendz[h] = mine[h].astype(jnp.bfloat16)
            rd(sendz.at[h], zrecv.at[h], _S_Z, h, nz).start()

            @pl.when(h > 0)
            def _():
                zreduce_and_gather(h - 1)

        @pl.when(jnp.logical_and(h == P - 1, ki == nk - 1))
        def _():
            zreduce_and_gather(P - 1)

            for p in range(P):
                rd(own_slice(p), own_slice(p), _S_X1, p, nx).wait()
                rd(own_slice(p), own_slice(p), _S_Y1, p, ny).wait()
                fx = out_ref.at[pl.ds(cy * CHUNK, HALF), pl.ds(p * cp, cp)]
                fy = out_ref.at[pl.ds(cx * CHUNK + HALF, HALF),
                                pl.ds(p * cp, cp)]
                rd(fx, fx, _S_X2, p, nx).start()
                rd(fy, fy, _S_Y2, p, ny).start()

            for p in range(P):
                fx = out_ref.at[pl.ds(cy * CHUNK, HALF), pl.ds(p * cp, cp)]
                fy = out_ref.at[pl.ds(cx * CHUNK + HALF, HALF),
                                pl.ds(p * cp, cp)]
                rd(fx, fx, _S_X2, p, nx).wait()
                rd(fy, fy, _S_Y2, p, ny).wait()

    grid_spec = pltpu.PrefetchScalarGridSpec(
        num_scalar_prefetch=1,
        grid=(P, nk),
        in_specs=[
            pl.BlockSpec((CHUNK, KB), lambda h, ki, c_ref: (c_ref[0], ki)),
            pl.BlockSpec((cp, KB), lambda h, ki, c_ref: (h, ki)),
        ],
        out_specs=pl.BlockSpec((m, d), lambda h, ki, c_ref: (0, 0)),
        scratch_shapes=[
            pltpu.VMEM((P, CHUNK, cp), jnp.float32),
            pltpu.VMEM((P, CHUNK, cp), jnp.bfloat16),
            pltpu.VMEM((P, CHUNK, cp), jnp.bfloat16),
            pltpu.SemaphoreType.DMA((5, P)),
            pltpu.SemaphoreType.DMA((5, P)),
        ],
    )
    return pl.pallas_call(
        body,
        grid_spec=grid_spec,
        out_shape=jax.ShapeDtypeStruct((m, d), jnp.bfloat16),
        compiler_params=pltpu.CompilerParams(
            collective_id=0,
            vmem_limit_bytes=100 * 1024 * 1024,
        ),
    )(jnp.array([c_out], dtype=jnp.int32), dy, W)
